# Initial kernel scaffold; baseline (speedup 1.0000x reference)
import jax
import jax.numpy as jnp
from jax.experimental import pallas as pl


def kernel(x, table, gamma, beta):
    raise NotImplementedError("write your pallas kernel here")


# speedup vs baseline: 1.6867x; 1.6867x over previous
import functools

import jax
import jax.numpy as jnp
import numpy as np
from jax import lax
from jax.experimental import pallas as pl
from jax.experimental.pallas import tpu as pltpu
from jax.experimental.pallas import tpu_sc as plsc

D = 32
EPS = 1e-5

NC = 2
NS = 16
NW = NC * NS

IDX_MINOR = 128
GATHERS_PER_CHUNK = 8
CHUNK = IDX_MINOR * GATHERS_PER_CHUNK


def _sc_gather(x_grouped, table):
    n_idx_rows = x_grouped.shape[1]
    n_per_w = n_idx_rows * IDX_MINOR
    n_chunks = n_per_w // CHUNK
    N = NW * n_per_w

    mesh = plsc.VectorSubcoreMesh(core_axis_name="c", subcore_axis_name="s")

    @functools.partial(
        pl.kernel,
        mesh=mesh,
        out_type=jax.ShapeDtypeStruct((N, D), jnp.float32),
        compiler_params=pltpu.CompilerParams(use_tc_tiling_on_sc=False),
        scratch_types=[
            pltpu.VMEM((n_idx_rows, IDX_MINOR), jnp.int32),
            pltpu.VMEM((CHUNK, D), jnp.float32),
            pltpu.SemaphoreType.DMA,
        ],
    )
    def k(x_hbm, table_hbm, out_hbm, idx_v, rows_v, sem):
        wid = lax.axis_index("c") * NS + lax.axis_index("s")
        base = wid * n_per_w
        pltpu.sync_copy(x_hbm.at[wid], idx_v)

        def chunk_body(c, carry):
            descs = []
            for j in range(GATHERS_PER_CHUNK):
                descs.append(
                    pltpu.async_copy(
                        table_hbm.at[idx_v.at[c * GATHERS_PER_CHUNK + j]],
                        rows_v.at[pl.ds(j * IDX_MINOR, IDX_MINOR)],
                        sem,
                    )
                )
            for d in descs:
                d.wait()
            pltpu.sync_copy(rows_v, out_hbm.at[pl.ds(base + c * CHUNK, CHUNK)])
            return carry

        lax.fori_loop(0, n_chunks, chunk_body, 0)

    return k(x_grouped, table)


def _tc_norm(z4, seg, gt, bt):
    N4 = z4.shape[0]
    BLK = 1024
    grid = (N4 // BLK,)

    def body(z_ref, seg_ref, g_ref, b_ref, o_ref):
        z = z_ref[...]
        s = seg_ref[...]
        s1 = jnp.dot(z, s, preferred_element_type=jnp.float32)
        s2 = jnp.dot(z * z, s, preferred_element_type=jnp.float32)
        mean = s1 * (1.0 / D)
        var = s2 * (1.0 / D) - mean * mean
        rstd = lax.rsqrt(var + EPS)
        o_ref[...] = jnp.maximum((z - mean) * rstd * g_ref[...] + b_ref[...], 0.0)

    return pl.pallas_call(
        body,
        grid=grid,
        in_specs=[
            pl.BlockSpec((BLK, 128), lambda i: (i, 0)),
            pl.BlockSpec((128, 128), lambda i: (0, 0)),
            pl.BlockSpec((1, 128), lambda i: (0, 0)),
            pl.BlockSpec((1, 128), lambda i: (0, 0)),
        ],
        out_specs=pl.BlockSpec((BLK, 128), lambda i: (i, 0)),
        out_shape=jax.ShapeDtypeStruct((N4, 128), jnp.float32),
    )(z4, seg, gt, bt)


def kernel(x, table, gamma, beta):
    B, L = x.shape
    N = B * L
    x_grouped = x.reshape(NW, N // (NW * IDX_MINOR), IDX_MINOR)
    g = _sc_gather(x_grouped, table)

    z4 = g.reshape(N // 4, 4 * D)
    seg = jnp.asarray(
        (np.arange(128)[:, None] // D) == (np.arange(128)[None, :] // D),
        dtype=jnp.float32,
    )
    gt = jnp.tile(gamma, 4).reshape(1, 128)
    bt = jnp.tile(beta, 4).reshape(1, 128)
    out = _tc_norm(z4, seg, gt, bt)
    return out.reshape(B, L, D)


# speedup vs baseline: 2.0643x; 1.2239x over previous
import functools

import jax
import jax.numpy as jnp
import numpy as np
from jax import lax
from jax.experimental import pallas as pl
from jax.experimental.pallas import tpu as pltpu
from jax.experimental.pallas import tpu_sc as plsc

D = 32
EPS = 1e-5

NC = 2
NS = 16
NW = NC * NS

IDX_MINOR = 128
GATHERS_PER_CHUNK = 8
CHUNK = IDX_MINOR * GATHERS_PER_CHUNK

KB = 2048
RB = KB // 4


def _sc_gather(x_grouped, table):
    n_idx_rows = x_grouped.shape[1]
    n_per_w = n_idx_rows * IDX_MINOR
    n_chunks = n_per_w // CHUNK
    N = NW * n_per_w

    mesh = plsc.VectorSubcoreMesh(core_axis_name="c", subcore_axis_name="s")

    @functools.partial(
        pl.kernel,
        mesh=mesh,
        out_type=jax.ShapeDtypeStruct((N, D), jnp.float32),
        compiler_params=pltpu.CompilerParams(use_tc_tiling_on_sc=False),
        scratch_types=[
            pltpu.VMEM((n_idx_rows, IDX_MINOR), jnp.int32),
            pltpu.VMEM((CHUNK, D), jnp.float32),
            pltpu.SemaphoreType.DMA,
        ],
    )
    def k(x_hbm, table_hbm, out_hbm, idx_v, rows_v, sem):
        wid = lax.axis_index("c") * NS + lax.axis_index("s")
        base = wid * n_per_w
        pltpu.sync_copy(x_hbm.at[wid], idx_v)

        def chunk_body(c, carry):
            descs = []
            for j in range(GATHERS_PER_CHUNK):
                descs.append(
                    pltpu.async_copy(
                        table_hbm.at[idx_v.at[c * GATHERS_PER_CHUNK + j]],
                        rows_v.at[pl.ds(j * IDX_MINOR, IDX_MINOR)],
                        sem,
                    )
                )
            for d in descs:
                d.wait()
            pltpu.sync_copy(rows_v, out_hbm.at[pl.ds(base + c * CHUNK, CHUNK)])
            return carry

        lax.fori_loop(0, n_chunks, chunk_body, 0)

    return k(x_grouped, table)


def _tc_norm_t(z4, seg, gt, bt, L, B):
    nkb = B // KB

    def body(z_ref, seg_ref, g_ref, b_ref, o_ref):
        z = z_ref[...]
        s = seg_ref[...]
        s1 = jnp.dot(z, s, preferred_element_type=jnp.float32)
        s2 = jnp.dot(z * z, s, preferred_element_type=jnp.float32)
        mean = s1 * (1.0 / D)
        var = s2 * (1.0 / D) - mean * mean
        rstd = lax.rsqrt(var + EPS)
        y = jnp.maximum((z - mean) * rstd * g_ref[...] + b_ref[...], 0.0)
        for sseg in range(4):
            o_ref[0, :, sseg * RB:(sseg + 1) * RB] = y[:, sseg * D:(sseg + 1) * D].T

    return pl.pallas_call(
        body,
        grid=(L, nkb),
        in_specs=[
            pl.BlockSpec((RB, 128), lambda l, kb: (l * nkb + kb, 0)),
            pl.BlockSpec((128, 128), lambda l, kb: (0, 0)),
            pl.BlockSpec((1, 128), lambda l, kb: (0, 0)),
            pl.BlockSpec((1, 128), lambda l, kb: (0, 0)),
        ],
        out_specs=pl.BlockSpec((1, D, KB), lambda l, kb: (l, 0, kb)),
        out_shape=jax.ShapeDtypeStruct((L, D, B), jnp.float32),
    )(z4, seg, gt, bt)


def kernel(x, table, gamma, beta):
    B, L = x.shape
    N = B * L
    nkb = B // KB
    xp = (
        x.T.reshape(L, nkb, 4, RB)
        .transpose(0, 1, 3, 2)
        .reshape(NW, N // (NW * IDX_MINOR), IDX_MINOR)
    )
    g = _sc_gather(xp, table)

    z4 = g.reshape(N // 4, 4 * D)
    seg = jnp.asarray(
        (np.arange(128)[:, None] // D) == (np.arange(128)[None, :] // D),
        dtype=jnp.float32,
    )
    gt = jnp.tile(gamma, 4).reshape(1, 128)
    bt = jnp.tile(beta, 4).reshape(1, 128)
    out_t = _tc_norm_t(z4, seg, gt, bt, L, B)
    return jnp.transpose(out_t, (2, 0, 1))
